# baseline (speedup 1.0000x reference)
import functools

import jax
import jax.numpy as jnp
from jax import lax
from jax.experimental import pallas as pl
from jax.experimental.pallas import tpu as pltpu
from jax.experimental.pallas import tpu_sc as plsc

N = 10000
E = 320000
H = 128
NC = 2
NS = 16
NW = NC * NS
EPW = E // NW
CHUNK = 80
NCHUNK = EPW // CHUNK
REC = 3 * CHUNK
BLK = 40
NBLK = N // BLK
BPT = -(-NBLK // NS)
GA = 3
OBLK = 80
ONBLK = N // OBLK
OBPT = -(-ONBLK // NS)


def _sc_agg(y, src, dst, ew):
    mesh = plsc.VectorSubcoreMesh(core_axis_name="c", subcore_axis_name="s")

    @functools.partial(
        pl.kernel,
        mesh=mesh,
        out_type=jax.ShapeDtypeStruct((NC, N, H), jnp.float32),
        scratch_types=(
            [pltpu.VMEM((CHUNK, H), jnp.float32) for _ in range(4)]
            + [pltpu.VMEM((CHUNK,), jnp.int32) for _ in range(4)]
            + [pltpu.VMEM((CHUNK,), jnp.int32) for _ in range(4)]
            + [pltpu.VMEM((CHUNK,), jnp.float32) for _ in range(4)]
            + [pltpu.VMEM((GA * 16,), jnp.int32) for _ in range(4)]
            + [pltpu.VMEM((CHUNK - GA * 16,), jnp.int32) for _ in range(4)]
            + [pltpu.VMEM((BLK, H), jnp.float32),
               pltpu.VMEM_SHARED((N, H), jnp.float32)]
            + [pltpu.SemaphoreType.DMA for _ in range(12)]
        ),
    )
    def agg_kernel(y_hbm, src_hbm, dst_hbm, ew_hbm, out_hbm,
                   r0, r1, r2, r3, sc0, sc1, sc2, sc3, dc0, dc1, dc2, dc3,
                   ec0, ec1, ec2, ec3, da0, da1, da2, da3,
                   db0, db1, db2, db3, zbuf, acc,
                   sg0, sg1, sg2, sg3, ss0, ss1, ss2, ss3,
                   si0, si1, si2, si3):
        ROWS = (r0, r1, r2, r3)
        SRCC = (sc0, sc1, sc2, sc3)
        DSTC = (dc0, dc1, dc2, dc3)
        EWC = (ec0, ec1, ec2, ec3)
        DSTDA = (da0, da1, da2, da3)
        DSTDB = (db0, db1, db2, db3)
        SG = (sg0, sg1, sg2, sg3)
        SS = (ss0, ss1, ss2, ss3)
        SI = (si0, si1, si2, si3)
        c = lax.axis_index("c")
        s = lax.axis_index("s")
        w = s * NC + c

        base = w * EPW

        def idx_issue(i, m):
            e0 = base + i * CHUNK
            pltpu.async_copy(src_hbm.at[pl.ds(e0, CHUNK)], SRCC[m], SI[m])
            pltpu.async_copy(dst_hbm.at[pl.ds(e0, CHUNK)], DSTC[m], SI[m])
            pltpu.async_copy(ew_hbm.at[pl.ds(e0, CHUNK)], EWC[m], SI[m])

        def idx_wait(m):
            d0 = pl.ds(0, CHUNK)
            pltpu.make_async_copy(src_hbm.at[d0], SRCC[m], SI[m]).wait()
            pltpu.make_async_copy(dst_hbm.at[d0], DSTC[m], SI[m]).wait()
            pltpu.make_async_copy(ew_hbm.at[d0], EWC[m], SI[m]).wait()

        def gather_issue(b):
            pltpu.async_copy(y_hbm.at[SRCC[b]], ROWS[b], SG[b])

        idx_issue(0, 0)
        idx_issue(1, 1)
        idx_issue(2, 2)
        idx_wait(0)
        gather_issue(0)
        idx_wait(1)
        gather_issue(1)

        zero16 = jnp.zeros((16,), jnp.float32)

        def zfill(t, _):
            zbuf[t // (H // 16), pl.ds((t % (H // 16)) * 16, 16)] = zero16
            return _
        lax.fori_loop(0, BLK * (H // 16), zfill, None)

        def zissue(t, _):
            blk = t * NS + s

            @pl.when(blk < NBLK)
            def _():
                pltpu.async_copy(zbuf, acc.at[pl.ds(blk * BLK, BLK)], ss0)
            return _
        lax.fori_loop(0, BPT, zissue, None)

        def zdrain(t, _):
            blk = t * NS + s

            @pl.when(blk < NBLK)
            def _():
                pltpu.make_async_copy(
                    zbuf, acc.at[pl.ds(0, BLK)], ss0).wait()
            return _
        lax.fori_loop(0, BPT, zdrain, None)
        plsc.subcore_barrier()

        def scatter_wait(b):
            pltpu.make_async_copy(ROWS[b].at[pl.ds(0, GA * 16)],
                                  acc.at[DSTDA[b]], SS[b]).wait()
            pltpu.make_async_copy(ROWS[b].at[pl.ds(GA * 16, CHUNK - GA * 16)],
                                  acc.at[DSTDB[b]], SS[b]).wait()

        def process(j, b, w_scat=True, p_idx=True, p_gather=True):
            rows = ROWS[b]
            if w_scat:
                scatter_wait((b + 2) % 4)
            if p_idx:
                idx_issue(j + 3, (b + 3) % 4)
            if p_gather:
                idx_wait((b + 2) % 4)
                gather_issue((b + 2) % 4)
            pltpu.make_async_copy(
                y_hbm.at[pl.ds(0, CHUNK)], rows, SG[b]).wait()

            ewc = EWC[b]

            def make_escale(dref, goff):
                def escale(g, _):
                    dref[pl.ds((g - goff) * 16, 16)] = (
                        DSTC[b][pl.ds(g * 16, 16)])
                    ew16 = ewc[pl.ds(g * 16, 16)]
                    for k in range(16):
                        w16 = lax.gather(
                            ew16, jnp.full((16, 1), k, jnp.int32),
                            dimension_numbers=lax.GatherDimensionNumbers(
                                offset_dims=(), collapsed_slice_dims=(0,),
                                start_index_map=(0,)),
                            slice_sizes=(1,),
                            mode=lax.GatherScatterMode.PROMISE_IN_BOUNDS)
                        e = g * 16 + k
                        for h in range(H // 16):
                            rows[e, pl.ds(h * 16, 16)] = (
                                rows[e, pl.ds(h * 16, 16)] * w16)
                    return _
                return escale
            lax.fori_loop(0, GA, make_escale(DSTDA[b], 0), None)
            pltpu.async_copy(rows.at[pl.ds(0, GA * 16)], acc.at[DSTDA[b]],
                             SS[b], add=True)
            lax.fori_loop(GA, CHUNK // 16, make_escale(DSTDB[b], GA), None)
            pltpu.async_copy(rows.at[pl.ds(GA * 16, CHUNK - GA * 16)],
                             acc.at[DSTDB[b]], SS[b], add=True)

        process(0, 0, w_scat=False)
        process(1, 1, w_scat=False)

        def quad_body(q, _):
            j0 = q * 4 + 2
            for b in range(4):
                process(j0 + b, (b + 2) % 4)
            return _
        lax.fori_loop(0, (NCHUNK - 5) // 4, quad_body, None)

        process(NCHUNK - 3, (NCHUNK - 3) % 4, p_idx=False)
        process(NCHUNK - 2, (NCHUNK - 2) % 4, p_idx=False, p_gather=False)
        process(NCHUNK - 1, (NCHUNK - 1) % 4, p_idx=False, p_gather=False)
        scatter_wait((NCHUNK - 2) % 4)
        scatter_wait((NCHUNK - 1) % 4)

        plsc.subcore_barrier()

        def oissue(t, _):
            blk = t * NS + s

            @pl.when(blk < ONBLK)
            def _():
                r0_ = blk * OBLK
                pltpu.async_copy(acc.at[pl.ds(r0_, OBLK)],
                                 out_hbm.at[c, pl.ds(r0_, OBLK)], sg0)
            return _
        lax.fori_loop(0, OBPT, oissue, None)

        def odrain(t, _):
            blk = t * NS + s

            @pl.when(blk < ONBLK)
            def _():
                pltpu.make_async_copy(
                    acc.at[pl.ds(0, OBLK)],
                    out_hbm.at[c, pl.ds(0, OBLK)], sg0).wait()
            return _
        lax.fori_loop(0, OBPT, odrain, None)

    return agg_kernel(y, src, dst, ew)


_BR = 5000
_GRID = N // _BR


def _rows_spec():
    return pl.BlockSpec((_BR, H), lambda i: (i, 0))


def _full_spec(shape):
    return pl.BlockSpec(shape, lambda i: tuple(0 for _ in shape))


def _tc_matmul(x, w):
    def body(x_ref, w_ref, y_ref):
        y_ref[...] = jnp.dot(x_ref[...], w_ref[...],
                             preferred_element_type=jnp.float32)
    return pl.pallas_call(
        body,
        grid=(_GRID,),
        in_specs=[_rows_spec(), _full_spec((H, H))],
        out_specs=_rows_spec(),
        out_shape=jax.ShapeDtypeStruct((N, H), jnp.float32),
    )(x, w)


def _tc_layer(x, agg, Wl, b, Wr_next):
    def body(x_ref, agg_ref, wl_ref, b_ref, wr_ref, xo_ref, yo_ref):
        h = (jnp.dot(x_ref[...], wl_ref[...],
                     preferred_element_type=jnp.float32)
             + agg_ref[0] + agg_ref[1] + b_ref[...])
        xn = jnp.maximum(h, 0.0)
        xo_ref[...] = xn
        yo_ref[...] = jnp.dot(xn, wr_ref[...],
                              preferred_element_type=jnp.float32)
    return pl.pallas_call(
        body,
        grid=(_GRID,),
        in_specs=[
            _rows_spec(),
            pl.BlockSpec((NC, _BR, H), lambda i: (0, i, 0)),
            _full_spec((H, H)),
            _full_spec((1, H)),
            _full_spec((H, H)),
        ],
        out_specs=[_rows_spec(), _rows_spec()],
        out_shape=[
            jax.ShapeDtypeStruct((N, H), jnp.float32),
            jax.ShapeDtypeStruct((N, H), jnp.float32),
        ],
    )(x, agg, Wl, b.reshape(1, H), Wr_next)


def _tc_final(x1, x2, agg3, Wl3, b3, A1, A2, A3, bl):
    def body(x1_ref, x2_ref, agg_ref, wl_ref, b_ref,
             a1_ref, a2_ref, a3_ref, bl_ref, o_ref):
        h = (jnp.dot(x2_ref[...], wl_ref[...],
                     preferred_element_type=jnp.float32)
             + agg_ref[0] + agg_ref[1] + b_ref[...])
        x3 = jnp.maximum(h, 0.0)
        t = (jnp.dot(x1_ref[...], a1_ref[...],
                     preferred_element_type=jnp.float32)
             + jnp.dot(x2_ref[...], a2_ref[...],
                       preferred_element_type=jnp.float32)
             + jnp.dot(x3, a3_ref[...],
                       preferred_element_type=jnp.float32)
             + bl_ref[...])
        mask = lax.broadcasted_iota(jnp.int32, (_BR, H), 1) < 7
        t = jnp.where(mask, t, -jnp.inf)
        m = jnp.max(t, axis=1, keepdims=True)
        lse = m + jnp.log(jnp.sum(jnp.exp(t - m), axis=1, keepdims=True))
        o_ref[...] = t - lse
    return pl.pallas_call(
        body,
        grid=(_GRID,),
        in_specs=[
            _rows_spec(),
            _rows_spec(),
            pl.BlockSpec((NC, _BR, H), lambda i: (0, i, 0)),
            _full_spec((H, H)),
            _full_spec((1, H)),
            _full_spec((H, H)),
            _full_spec((H, H)),
            _full_spec((H, H)),
            _full_spec((1, H)),
        ],
        out_specs=_rows_spec(),
        out_shape=jax.ShapeDtypeStruct((N, H), jnp.float32),
    )(x1, x2, agg3, Wl3, b3.reshape(1, H), A1, A2, A3, bl)


def kernel(x0, edge_index, edge_weight, Wl1, Wr1, b1, Wl2, Wr2, b2,
           Wl3, Wr3, b3, Wlin, blin):
    src = edge_index[0]
    dst = edge_index[1]

    y0 = _tc_matmul(x0, Wr1)
    agg1 = _sc_agg(y0, src, dst, edge_weight)
    x1, y1 = _tc_layer(x0, agg1, Wl1, b1, Wr2)

    agg2 = _sc_agg(y1, src, dst, edge_weight)
    x2, y2 = _tc_layer(x1, agg2, Wl2, b2, Wr3)

    agg3 = _sc_agg(y2, src, dst, edge_weight)

    C = Wlin.shape[1]
    A = jnp.zeros((3 * H, H), jnp.float32).at[:, :C].set(Wlin)
    bl = jnp.zeros((1, H), jnp.float32).at[0, :C].set(blin)
    out = _tc_final(x1, x2, agg3, Wl3, b3, A[:H], A[H:2 * H], A[2 * H:], bl)
    return out[:, :C]


# speedup vs baseline: 1.0133x; 1.0133x over previous
import functools

import jax
import jax.numpy as jnp
from jax import lax
from jax.experimental import pallas as pl
from jax.experimental.pallas import tpu as pltpu
from jax.experimental.pallas import tpu_sc as plsc

N = 10000
E = 320000
H = 128
NC = 2
NS = 16
NW = NC * NS
EPW = E // NW
CHUNK = 80
NCHUNK = EPW // CHUNK
REC = 3 * CHUNK
BLK = 40
NBLK = N // BLK
BPT = -(-NBLK // NS)
OBLK = 80
ONBLK = N // OBLK
OBPT = -(-ONBLK // NS)


def _sc_agg(y, src, dst, ew):
    mesh = plsc.VectorSubcoreMesh(core_axis_name="c", subcore_axis_name="s")

    @functools.partial(
        pl.kernel,
        mesh=mesh,
        out_type=jax.ShapeDtypeStruct((NC, N, H), jnp.float32),
        scratch_types=(
            [pltpu.VMEM((CHUNK, H), jnp.float32) for _ in range(4)]
            + [pltpu.VMEM((CHUNK,), jnp.int32) for _ in range(4)]
            + [pltpu.VMEM((CHUNK,), jnp.int32) for _ in range(4)]
            + [pltpu.VMEM((CHUNK,), jnp.float32) for _ in range(4)]
            + [pltpu.VMEM((CHUNK,), jnp.int32) for _ in range(4)]
            + [pltpu.VMEM((BLK, H), jnp.float32),
               pltpu.VMEM_SHARED((N, H), jnp.float32)]
            + [pltpu.SemaphoreType.DMA for _ in range(12)]
        ),
    )
    def agg_kernel(y_hbm, src_hbm, dst_hbm, ew_hbm, out_hbm,
                   r0, r1, r2, r3, sc0, sc1, sc2, sc3, dc0, dc1, dc2, dc3,
                   ec0, ec1, ec2, ec3, dd0, dd1, dd2, dd3, zbuf, acc,
                   sg0, sg1, sg2, sg3, ss0, ss1, ss2, ss3,
                   si0, si1, si2, si3):
        ROWS = (r0, r1, r2, r3)
        SRCC = (sc0, sc1, sc2, sc3)
        DSTC = (dc0, dc1, dc2, dc3)
        EWC = (ec0, ec1, ec2, ec3)
        DSTD = (dd0, dd1, dd2, dd3)
        SG = (sg0, sg1, sg2, sg3)
        SS = (ss0, ss1, ss2, ss3)
        SI = (si0, si1, si2, si3)
        c = lax.axis_index("c")
        s = lax.axis_index("s")
        w = s * NC + c

        base = w * EPW

        def idx_issue(i, m):
            e0 = base + i * CHUNK
            pltpu.async_copy(src_hbm.at[pl.ds(e0, CHUNK)], SRCC[m], SI[m])
            pltpu.async_copy(dst_hbm.at[pl.ds(e0, CHUNK)], DSTC[m], SI[m])
            pltpu.async_copy(ew_hbm.at[pl.ds(e0, CHUNK)], EWC[m], SI[m])

        def idx_wait(m):
            d0 = pl.ds(0, CHUNK)
            pltpu.make_async_copy(src_hbm.at[d0], SRCC[m], SI[m]).wait()
            pltpu.make_async_copy(dst_hbm.at[d0], DSTC[m], SI[m]).wait()
            pltpu.make_async_copy(ew_hbm.at[d0], EWC[m], SI[m]).wait()

        def gather_issue(b):
            pltpu.async_copy(y_hbm.at[SRCC[b]], ROWS[b], SG[b])

        idx_issue(0, 0)
        idx_issue(1, 1)
        idx_issue(2, 2)
        idx_wait(0)
        gather_issue(0)
        idx_wait(1)
        gather_issue(1)

        zero16 = jnp.zeros((16,), jnp.float32)

        def zfill(t, _):
            zbuf[t // (H // 16), pl.ds((t % (H // 16)) * 16, 16)] = zero16
            return _
        lax.fori_loop(0, BLK * (H // 16), zfill, None)

        def zissue(t, _):
            blk = t * NS + s

            @pl.when(blk < NBLK)
            def _():
                pltpu.async_copy(zbuf, acc.at[pl.ds(blk * BLK, BLK)], ss0)
            return _
        lax.fori_loop(0, BPT, zissue, None)

        def zdrain(t, _):
            blk = t * NS + s

            @pl.when(blk < NBLK)
            def _():
                pltpu.make_async_copy(
                    zbuf, acc.at[pl.ds(0, BLK)], ss0).wait()
            return _
        lax.fori_loop(0, BPT, zdrain, None)
        plsc.subcore_barrier()

        def scatter_wait(b):
            pltpu.make_async_copy(ROWS[b], acc.at[DSTD[b]], SS[b]).wait()

        def process(j, b, w_scat=True, p_idx=True, p_gather=True):
            rows = ROWS[b]
            if w_scat:
                scatter_wait((b + 2) % 4)
            if p_idx:
                idx_issue(j + 3, (b + 3) % 4)
            if p_gather:
                idx_wait((b + 2) % 4)
                gather_issue((b + 2) % 4)
            pltpu.make_async_copy(
                y_hbm.at[pl.ds(0, CHUNK)], rows, SG[b]).wait()

            ewc = EWC[b]

            def escale(g, _):
                DSTD[b][pl.ds(g * 16, 16)] = DSTC[b][pl.ds(g * 16, 16)]
                ew16 = ewc[pl.ds(g * 16, 16)]
                for k in range(16):
                    w16 = lax.gather(
                        ew16, jnp.full((16, 1), k, jnp.int32),
                        dimension_numbers=lax.GatherDimensionNumbers(
                            offset_dims=(), collapsed_slice_dims=(0,),
                            start_index_map=(0,)),
                        slice_sizes=(1,),
                        mode=lax.GatherScatterMode.PROMISE_IN_BOUNDS)
                    e = g * 16 + k
                    for h in range(H // 16):
                        rows[e, pl.ds(h * 16, 16)] = (
                            rows[e, pl.ds(h * 16, 16)] * w16)
                return _
            lax.fori_loop(0, CHUNK // 16, escale, None)

            pltpu.async_copy(rows, acc.at[DSTD[b]], SS[b], add=True)

        process(0, 0, w_scat=False)
        process(1, 1, w_scat=False)

        def quad_body(q, _):
            j0 = q * 4 + 2
            for b in range(4):
                process(j0 + b, (b + 2) % 4)
            return _
        lax.fori_loop(0, (NCHUNK - 5) // 4, quad_body, None)

        process(NCHUNK - 3, (NCHUNK - 3) % 4, p_idx=False)
        process(NCHUNK - 2, (NCHUNK - 2) % 4, p_idx=False, p_gather=False)
        process(NCHUNK - 1, (NCHUNK - 1) % 4, p_idx=False, p_gather=False)
        scatter_wait((NCHUNK - 2) % 4)
        scatter_wait((NCHUNK - 1) % 4)

        plsc.subcore_barrier()

        def oissue(t, _):
            blk = t * NS + s

            @pl.when(blk < ONBLK)
            def _():
                r0_ = blk * OBLK
                pltpu.async_copy(acc.at[pl.ds(r0_, OBLK)],
                                 out_hbm.at[c, pl.ds(r0_, OBLK)], sg0)
            return _
        lax.fori_loop(0, OBPT, oissue, None)

        def odrain(t, _):
            blk = t * NS + s

            @pl.when(blk < ONBLK)
            def _():
                pltpu.make_async_copy(
                    acc.at[pl.ds(0, OBLK)],
                    out_hbm.at[c, pl.ds(0, OBLK)], sg0).wait()
            return _
        lax.fori_loop(0, OBPT, odrain, None)

    return agg_kernel(y, src, dst, ew)


_BR = 5000
_GRID = N // _BR


def _rows_spec():
    return pl.BlockSpec((_BR, H), lambda i: (i, 0))


def _full_spec(shape):
    return pl.BlockSpec(shape, lambda i: tuple(0 for _ in shape))


def _tc_matmul(x, w):
    def body(x_ref, w_ref, y_ref):
        y_ref[...] = jnp.dot(x_ref[...], w_ref[...],
                             preferred_element_type=jnp.float32)
    return pl.pallas_call(
        body,
        grid=(_GRID,),
        in_specs=[_rows_spec(), _full_spec((H, H))],
        out_specs=_rows_spec(),
        out_shape=jax.ShapeDtypeStruct((N, H), jnp.float32),
    )(x, w)


def _tc_layer(x, agg, Wl, b, Wr_next):
    def body(x_ref, agg_ref, wl_ref, b_ref, wr_ref, xo_ref, yo_ref):
        h = (jnp.dot(x_ref[...], wl_ref[...],
                     preferred_element_type=jnp.float32)
             + agg_ref[0] + agg_ref[1] + b_ref[...])
        xn = jnp.maximum(h, 0.0)
        xo_ref[...] = xn
        yo_ref[...] = jnp.dot(xn, wr_ref[...],
                              preferred_element_type=jnp.float32)
    return pl.pallas_call(
        body,
        grid=(_GRID,),
        in_specs=[
            _rows_spec(),
            pl.BlockSpec((NC, _BR, H), lambda i: (0, i, 0)),
            _full_spec((H, H)),
            _full_spec((1, H)),
            _full_spec((H, H)),
        ],
        out_specs=[_rows_spec(), _rows_spec()],
        out_shape=[
            jax.ShapeDtypeStruct((N, H), jnp.float32),
            jax.ShapeDtypeStruct((N, H), jnp.float32),
        ],
    )(x, agg, Wl, b.reshape(1, H), Wr_next)


def _tc_final(x1, x2, agg3, Wl3, b3, A1, A2, A3, bl):
    def body(x1_ref, x2_ref, agg_ref, wl_ref, b_ref,
             a1_ref, a2_ref, a3_ref, bl_ref, o_ref):
        h = (jnp.dot(x2_ref[...], wl_ref[...],
                     preferred_element_type=jnp.float32)
             + agg_ref[0] + agg_ref[1] + b_ref[...])
        x3 = jnp.maximum(h, 0.0)
        t = (jnp.dot(x1_ref[...], a1_ref[...],
                     preferred_element_type=jnp.float32)
             + jnp.dot(x2_ref[...], a2_ref[...],
                       preferred_element_type=jnp.float32)
             + jnp.dot(x3, a3_ref[...],
                       preferred_element_type=jnp.float32)
             + bl_ref[...])
        mask = lax.broadcasted_iota(jnp.int32, (_BR, H), 1) < 7
        t = jnp.where(mask, t, -jnp.inf)
        m = jnp.max(t, axis=1, keepdims=True)
        lse = m + jnp.log(jnp.sum(jnp.exp(t - m), axis=1, keepdims=True))
        o_ref[...] = t - lse
    return pl.pallas_call(
        body,
        grid=(_GRID,),
        in_specs=[
            _rows_spec(),
            _rows_spec(),
            pl.BlockSpec((NC, _BR, H), lambda i: (0, i, 0)),
            _full_spec((H, H)),
            _full_spec((1, H)),
            _full_spec((H, H)),
            _full_spec((H, H)),
            _full_spec((H, H)),
            _full_spec((1, H)),
        ],
        out_specs=_rows_spec(),
        out_shape=jax.ShapeDtypeStruct((N, H), jnp.float32),
    )(x1, x2, agg3, Wl3, b3.reshape(1, H), A1, A2, A3, bl)


def kernel(x0, edge_index, edge_weight, Wl1, Wr1, b1, Wl2, Wr2, b2,
           Wl3, Wr3, b3, Wlin, blin):
    src = edge_index[0]
    dst = edge_index[1]

    y0 = _tc_matmul(x0, Wr1)
    agg1 = _sc_agg(y0, src, dst, edge_weight)
    x1, y1 = _tc_layer(x0, agg1, Wl1, b1, Wr2)

    agg2 = _sc_agg(y1, src, dst, edge_weight)
    x2, y2 = _tc_layer(x1, agg2, Wl2, b2, Wr3)

    agg3 = _sc_agg(y2, src, dst, edge_weight)

    C = Wlin.shape[1]
    A = jnp.zeros((3 * H, H), jnp.float32).at[:, :C].set(Wlin)
    bl = jnp.zeros((1, H), jnp.float32).at[0, :C].set(blin)
    out = _tc_final(x1, x2, agg3, Wl3, b3, A[:H], A[H:2 * H], A[2 * H:], bl)
    return out[:, :C]
